# baseline (speedup 1.0000x reference)
import functools
import math

import jax
import jax.numpy as jnp
from jax import lax
from jax.experimental import pallas as pl
from jax.experimental.pallas import tpu as pltpu
from jax.experimental.pallas import tpu_sc as plsc

_MARGIN = 0

_CW = 2944
_POS = _CW // 16
_SC_ROWS = 256


def _sc_gather(labels, indexes):
    b = indexes.shape[0]
    info = plsc.get_sparse_core_info()
    num_cores = info.num_cores
    nw = info.num_cores * info.num_subcores
    bpw = b // nw

    mesh = plsc.VectorSubcoreMesh(core_axis_name="c", subcore_axis_name="s")

    @functools.partial(
        pl.kernel,
        mesh=mesh,
        out_type=jax.ShapeDtypeStruct((b,), jnp.int32),
        scratch_types=[
            pltpu.VMEM((bpw,), jnp.int32),
            pltpu.VMEM((bpw,), jnp.int32),
            pltpu.SemaphoreType.DMA,
        ],
    )
    def sc_kernel(labels_hbm, idx_hbm, y_hbm, idx_v, y_v, sem):
        wid = lax.axis_index("s") * num_cores + lax.axis_index("c")
        base = wid * bpw
        pltpu.sync_copy(idx_hbm.at[pl.ds(base, bpw)], idx_v)
        pltpu.async_copy(labels_hbm.at[idx_v], y_v, sem).wait()
        pltpu.sync_copy(y_v, y_hbm.at[pl.ds(base, bpw)])

    return sc_kernel(labels, indexes)


def _sc_stream(x, labels_pad, indexes, row_base, s_rows):
    b, n = x.shape
    n_pad = labels_pad.shape[0]
    assert n_pad % _CW == 0
    n_chunks = n_pad // _CW
    pad_lanes = n_pad - n
    assert pad_lanes % 16 == 0 and pad_lanes < _CW
    info = plsc.get_sparse_core_info()
    nc = info.num_cores
    nw = info.num_cores * info.num_subcores
    rpw = s_rows // nw
    assert rpw % 8 == 0
    n_groups = rpw // 8
    rpw16 = max(rpw, 16)

    mesh = plsc.VectorSubcoreMesh(core_axis_name="c", subcore_axis_name="s")

    @functools.partial(
        pl.kernel,
        mesh=mesh,
        out_type=(
            jax.ShapeDtypeStruct((s_rows * 16,), jnp.float32),
            jax.ShapeDtypeStruct((s_rows * 16,), jnp.float32),
        ),
        scratch_types=[
            pltpu.VMEM((8, _CW), jnp.float32),
            pltpu.VMEM((8, _CW), jnp.float32),
            pltpu.VMEM((_CW,), jnp.int32),
            pltpu.VMEM((_CW,), jnp.int32),
            pltpu.VMEM((rpw16,), jnp.int32),
            pltpu.VMEM((rpw16,), jnp.int32),
            pltpu.VMEM((rpw * 16,), jnp.float32),
            pltpu.VMEM((rpw * 16,), jnp.float32),
            pltpu.SemaphoreType.DMA,
            pltpu.SemaphoreType.DMA,
            pltpu.SemaphoreType.DMA,
            pltpu.SemaphoreType.DMA,
        ],
    )
    def sc_kernel(x_hbm, lab_hbm, idx_hbm, p_hbm, z_hbm,
                  xb0, xb1, lb0, lb1, idx_v, y_v, p_acc, z_acc,
                  sem_x0, sem_x1, sem_l0, sem_l1):
        wid = lax.axis_index("s") * nc + lax.axis_index("c")
        row0 = row_base + wid * rpw
        out0 = wid * rpw
        pltpu.sync_copy(idx_hbm.at[pl.ds(row0, rpw)],
                        idx_v.at[pl.ds(0, rpw)])
        if rpw < 16:
            lane = lax.iota(jnp.int32, 16)
            iv = idx_v[pl.ds(0, 16)]
            idx_v[pl.ds(0, 16)] = jnp.where(lane < rpw, iv, 0)
        pltpu.async_copy(lab_hbm.at[idx_v], y_v, sem_l0).wait()

        iota = lax.iota(jnp.int32, 16)
        xbufs = (xb0, xb1)
        lbufs = (lb0, lb1)
        xsems = (sem_x0, sem_x1)
        lsems = (sem_l0, sem_l1)

        def do_group(g):
            grow = row0 + g * 8

            def start_dma(c, buf_i):
                pltpu.async_copy(
                    x_hbm.at[pl.ds(grow, 8), pl.ds(c * _CW, _CW)],
                    xbufs[buf_i], xsems[buf_i])
                pltpu.async_copy(
                    lab_hbm.at[pl.ds(c * _CW, _CW)],
                    lbufs[buf_i], lsems[buf_i])

            def wait_dma(buf_i):
                pltpu.make_async_copy(
                    x_hbm.at[pl.ds(grow, 8), pl.ds(0, _CW)],
                    xbufs[buf_i], xsems[buf_i]).wait()
                pltpu.make_async_copy(
                    lab_hbm.at[pl.ds(0, _CW)],
                    lbufs[buf_i], lsems[buf_i]).wait()

            yv16 = y_v[pl.ds((g // 2) * 16, 16)]
            iv16 = idx_v[pl.ds((g // 2) * 16, 16)]
            lo = (g % 2) * 8
            ys = [jnp.full((16,), yv16[lo + r], jnp.int32) for r in range(8)]
            ix = [jnp.full((16,), iv16[lo + r], jnp.int32) for r in range(8)]

            def do_chunk(c, buf_i):
                wait_dma(buf_i)
                xb = xbufs[buf_i]
                lb = lbufs[buf_i]

                @pl.when(c == n_chunks - 1)
                def _patch_pad():
                    ninf = jnp.full((16,), -jnp.inf, jnp.float32)
                    for r in range(8):
                        for t in range(pad_lanes // 16):
                            xb[r, pl.ds(_CW - pad_lanes + t * 16, 16)] = ninf

                accs0 = tuple(jnp.zeros((16,), jnp.float32)
                              for _ in range(16))

                def pos_step(i, carry):
                    accs = carry
                    colv = c * _CW + i * 16 + iota
                    lv = lb[pl.ds(i * 16, 16)]
                    out = []
                    for r in range(8):
                        az, ap = accs[2 * r], accs[2 * r + 1]
                        e = jnp.exp(xb[r, pl.ds(i * 16, 16)])
                        e = jnp.where(colv != ix[r], e, 0.0)
                        az = az + e
                        ap = ap + jnp.where(lv == ys[r], e, 0.0)
                        out.extend((az, ap))
                    return tuple(out)

                accs = lax.fori_loop(0, _POS, pos_step, accs0)
                for r in range(8):
                    o = pl.ds((g * 8 + r) * 16, 16)
                    z_acc[o] = z_acc[o] + accs[2 * r]
                    p_acc[o] = p_acc[o] + accs[2 * r + 1]

            for r in range(8):
                o = pl.ds((g * 8 + r) * 16, 16)
                z_acc[o] = jnp.zeros((16,), jnp.float32)
                p_acc[o] = jnp.zeros((16,), jnp.float32)
            start_dma(0, 0)

            def two_chunks(t, _):
                c0 = t * 2

                @pl.when(c0 + 1 < n_chunks)
                def _():
                    start_dma(c0 + 1, 1)
                do_chunk(c0, 0)

                @pl.when(c0 + 1 < n_chunks)
                def _():
                    @pl.when(c0 + 2 < n_chunks)
                    def _():
                        start_dma(c0 + 2, 0)
                    do_chunk(c0 + 1, 1)
                return 0

            lax.fori_loop(0, (n_chunks + 1) // 2, two_chunks, 0)

            pltpu.sync_copy(
                p_acc.at[pl.ds(g * 128, 128)],
                p_hbm.at[pl.ds((out0 + g * 8) * 16, 128)])
            pltpu.sync_copy(
                z_acc.at[pl.ds(g * 128, 128)],
                z_hbm.at[pl.ds((out0 + g * 8) * 16, 128)])

        for g in range(n_groups):
            do_group(g)

    return sc_kernel(x, labels_pad, indexes)


def _tc_body(n_cols, n_blocks, blk_w, bt,
             xa_ref, xb_ref, lab_ref, y_ref, idx_ref,
             p_out, z_out, p_acc, z_acc):
    j = pl.program_id(0)

    @pl.when(j == 0)
    def _init():
        p_acc[...] = jnp.zeros_like(p_acc)
        z_acc[...] = jnp.zeros_like(z_acc)

    half = bt // 2
    col = j * blk_w + lax.broadcasted_iota(jnp.int32, (1, blk_w), 1)
    valid = col < n_cols
    lab = lab_ref[...]
    for x_ref, r0 in ((xa_ref, 0), (xb_ref, half)):
        rows = pl.ds(r0, half)
        e = jnp.exp(x_ref[...])
        e = jnp.where(valid & (col != idx_ref[rows, :]), e, 0.0)
        same = lab == y_ref[rows, :]
        z_acc[rows, :] += jnp.sum(e, axis=1, keepdims=True)
        p_acc[rows, :] += jnp.sum(jnp.where(same, e, 0.0), axis=1,
                                  keepdims=True)

    @pl.when(j == n_blocks - 1)
    def _emit():
        p_out[...] = p_acc[...]
        z_out[...] = z_acc[...]


def _tc_main(x_full, bt, labels2d, y2d, idx2d, blk_w=4096):
    n_cols = x_full.shape[1]
    half = bt // 2
    n_blocks = pl.cdiv(n_cols, blk_w)
    outp = jax.ShapeDtypeStruct((bt, 1), jnp.float32)
    body = functools.partial(_tc_body, n_cols, n_blocks, blk_w, bt)
    return pl.pallas_call(
        body,
        grid=(n_blocks,),
        in_specs=[
            pl.BlockSpec((half, blk_w), lambda j: (0, j)),
            pl.BlockSpec((half, blk_w), lambda j: (1, j)),
            pl.BlockSpec((1, blk_w), lambda j: (0, j)),
            pl.BlockSpec((bt, 1), lambda j: (0, 0)),
            pl.BlockSpec((bt, 1), lambda j: (0, 0)),
        ],
        out_specs=[
            pl.BlockSpec((bt, 1), lambda j: (0, 0)),
            pl.BlockSpec((bt, 1), lambda j: (0, 0)),
        ],
        out_shape=[outp, outp],
        scratch_shapes=[
            pltpu.VMEM((bt, 1), jnp.float32),
            pltpu.VMEM((bt, 1), jnp.float32),
        ],
        compiler_params=pltpu.CompilerParams(
            dimension_semantics=("arbitrary",),
        ),
    )(x_full, x_full, labels2d, y2d, idx2d)


def _tc_finalize(p_t, z_t, p16_s, z16_s, batch):
    out11 = jax.ShapeDtypeStruct((1, 1), jnp.float32)

    def stats(p, z):
        prob = p / z
        nzm = prob != 0.0
        logp = jnp.where(nzm, jnp.log(jnp.where(nzm, prob, 1.0)), 0.0)
        return jnp.sum(logp), jnp.min(p), jnp.sum(p)

    def body(pt_ref, zt_ref, ps_ref, zs_ref, loss_ref, min_ref, mean_ref):
        scale = 1.0 / math.exp(_MARGIN)
        pt = pt_ref[...] * scale
        zt = zt_ref[...] - pt_ref[...] + pt
        ps = jnp.sum(ps_ref[...], axis=1, keepdims=True) * scale
        zs = (jnp.sum(zs_ref[...], axis=1, keepdims=True)
              - jnp.sum(ps_ref[...], axis=1, keepdims=True) + ps)
        lt, mt, st = stats(pt, zt)
        ls, ms, ss = stats(ps, zs)
        loss_ref[...] = jnp.full((1, 1), -1.0 / batch) * (lt + ls)
        min_ref[...] = jnp.full((1, 1), 1.0) * jnp.minimum(mt, ms)
        mean_ref[...] = jnp.full((1, 1), 1.0 / batch) * (st + ss)

    return pl.pallas_call(
        body,
        out_shape=[out11, out11, out11],
    )(p_t, z_t, p16_s, z16_s)


def kernel(x, features, labels, indexes):
    del features
    batch, n_cols = x.shape
    s_rows = _SC_ROWS
    bt = batch - s_rows
    n_pad = _CW * ((n_cols + _CW - 1) // _CW)
    labels_pad = jnp.pad(labels, (0, n_pad - n_cols), constant_values=-1)
    y = _sc_gather(labels, indexes)
    p_t, z_t = _tc_main(
        x, bt,
        labels.reshape(1, n_cols),
        y[:bt].reshape(bt, 1),
        indexes[:bt].reshape(bt, 1),
    )
    p16_s, z16_s = _sc_stream(x, labels_pad, indexes, bt, s_rows)
    loss, pmin, pmean = _tc_finalize(
        p_t, z_t,
        p16_s.reshape(s_rows, 16), z16_s.reshape(s_rows, 16),
        batch)
    return (loss[0, 0], pmin[0, 0], pmean[0, 0])


# speedup vs baseline: 1.0760x; 1.0760x over previous
import functools
import math

import jax
import jax.numpy as jnp
from jax import lax
from jax.experimental import pallas as pl
from jax.experimental.pallas import tpu as pltpu
from jax.experimental.pallas import tpu_sc as plsc

_MARGIN = 0

_CW = 2944
_POS = _CW // 16
_SC_ROWS = 256


def _sc_gather(labels, indexes):
    b = indexes.shape[0]
    info = plsc.get_sparse_core_info()
    num_cores = info.num_cores
    nw = info.num_cores * info.num_subcores
    bpw = b // nw

    mesh = plsc.VectorSubcoreMesh(core_axis_name="c", subcore_axis_name="s")

    @functools.partial(
        pl.kernel,
        mesh=mesh,
        out_type=jax.ShapeDtypeStruct((b,), jnp.int32),
        scratch_types=[
            pltpu.VMEM((bpw,), jnp.int32),
            pltpu.VMEM((bpw,), jnp.int32),
            pltpu.SemaphoreType.DMA,
        ],
    )
    def sc_kernel(labels_hbm, idx_hbm, y_hbm, idx_v, y_v, sem):
        wid = lax.axis_index("s") * num_cores + lax.axis_index("c")
        base = wid * bpw
        pltpu.sync_copy(idx_hbm.at[pl.ds(base, bpw)], idx_v)
        pltpu.async_copy(labels_hbm.at[idx_v], y_v, sem).wait()
        pltpu.sync_copy(y_v, y_hbm.at[pl.ds(base, bpw)])

    return sc_kernel(labels, indexes)


def _sc_stream(x, labels_pad, indexes, row_base, s_rows):
    b, n = x.shape
    n_pad = labels_pad.shape[0]
    assert n_pad % _CW == 0
    n_chunks = n_pad // _CW
    pad_lanes = n_pad - n
    assert pad_lanes % 16 == 0 and pad_lanes < _CW
    info = plsc.get_sparse_core_info()
    nc = info.num_cores
    nw = info.num_cores * info.num_subcores
    rpw = s_rows // nw
    assert rpw % 8 == 0
    n_groups = rpw // 8
    rpw16 = max(rpw, 16)

    mesh = plsc.VectorSubcoreMesh(core_axis_name="c", subcore_axis_name="s")

    @functools.partial(
        pl.kernel,
        mesh=mesh,
        out_type=(
            jax.ShapeDtypeStruct((s_rows * 16,), jnp.float32),
            jax.ShapeDtypeStruct((s_rows * 16,), jnp.float32),
        ),
        scratch_types=[
            pltpu.VMEM((8, _CW), jnp.float32),
            pltpu.VMEM((8, _CW), jnp.float32),
            pltpu.VMEM((_CW,), jnp.int32),
            pltpu.VMEM((_CW,), jnp.int32),
            pltpu.VMEM((rpw16,), jnp.int32),
            pltpu.VMEM((rpw16,), jnp.int32),
            pltpu.VMEM((rpw * 16,), jnp.float32),
            pltpu.VMEM((rpw * 16,), jnp.float32),
            pltpu.SemaphoreType.DMA,
            pltpu.SemaphoreType.DMA,
            pltpu.SemaphoreType.DMA,
            pltpu.SemaphoreType.DMA,
        ],
    )
    def sc_kernel(x_hbm, lab_hbm, idx_hbm, p_hbm, z_hbm,
                  xb0, xb1, lb0, lb1, idx_v, y_v, p_acc, z_acc,
                  sem_x0, sem_x1, sem_l0, sem_l1):
        wid = lax.axis_index("s") * nc + lax.axis_index("c")
        row0 = row_base + wid * rpw
        out0 = wid * rpw
        pltpu.sync_copy(idx_hbm.at[pl.ds(row0, rpw)],
                        idx_v.at[pl.ds(0, rpw)])
        if rpw < 16:
            lane = lax.iota(jnp.int32, 16)
            iv = idx_v[pl.ds(0, 16)]
            idx_v[pl.ds(0, 16)] = jnp.where(lane < rpw, iv, 0)
        pltpu.async_copy(lab_hbm.at[idx_v], y_v, sem_l0).wait()

        iota = lax.iota(jnp.int32, 16)
        xbufs = (xb0, xb1)
        lbufs = (lb0, lb1)
        xsems = (sem_x0, sem_x1)
        lsems = (sem_l0, sem_l1)

        def do_group(g):
            grow = row0 + g * 8

            def start_dma(c, buf_i):
                pltpu.async_copy(
                    x_hbm.at[pl.ds(grow, 8), pl.ds(c * _CW, _CW)],
                    xbufs[buf_i], xsems[buf_i])
                pltpu.async_copy(
                    lab_hbm.at[pl.ds(c * _CW, _CW)],
                    lbufs[buf_i], lsems[buf_i])

            def wait_dma(buf_i):
                pltpu.make_async_copy(
                    x_hbm.at[pl.ds(grow, 8), pl.ds(0, _CW)],
                    xbufs[buf_i], xsems[buf_i]).wait()
                pltpu.make_async_copy(
                    lab_hbm.at[pl.ds(0, _CW)],
                    lbufs[buf_i], lsems[buf_i]).wait()

            yv16 = y_v[pl.ds((g // 2) * 16, 16)]
            iv16 = idx_v[pl.ds((g // 2) * 16, 16)]
            lo = (g % 2) * 8
            ys = [jnp.full((16,), yv16[lo + r], jnp.int32) for r in range(8)]
            ix = [jnp.full((16,), iv16[lo + r], jnp.int32) for r in range(8)]

            def do_chunk(c, buf_i):
                wait_dma(buf_i)
                xb = xbufs[buf_i]
                lb = lbufs[buf_i]

                @pl.when(c == n_chunks - 1)
                def _patch_pad():
                    ninf = jnp.full((16,), -jnp.inf, jnp.float32)
                    for r in range(8):
                        for t in range(pad_lanes // 16):
                            xb[r, pl.ds(_CW - pad_lanes + t * 16, 16)] = ninf

                accs0 = tuple(jnp.zeros((16,), jnp.float32)
                              for _ in range(16))

                def pos_step(i, carry):
                    accs = carry
                    colv = c * _CW + i * 16 + iota
                    lv = lb[pl.ds(i * 16, 16)]
                    out = []
                    for r in range(8):
                        az, ap = accs[2 * r], accs[2 * r + 1]
                        e = jnp.exp(xb[r, pl.ds(i * 16, 16)])
                        e = jnp.where(colv != ix[r], e, 0.0)
                        az = az + e
                        ap = ap + jnp.where(lv == ys[r], e, 0.0)
                        out.extend((az, ap))
                    return tuple(out)

                accs = lax.fori_loop(0, _POS, pos_step, accs0)
                for r in range(8):
                    o = pl.ds((g * 8 + r) * 16, 16)
                    z_acc[o] = z_acc[o] + accs[2 * r]
                    p_acc[o] = p_acc[o] + accs[2 * r + 1]

            for r in range(8):
                o = pl.ds((g * 8 + r) * 16, 16)
                z_acc[o] = jnp.zeros((16,), jnp.float32)
                p_acc[o] = jnp.zeros((16,), jnp.float32)
            start_dma(0, 0)

            def two_chunks(t, _):
                c0 = t * 2

                @pl.when(c0 + 1 < n_chunks)
                def _():
                    start_dma(c0 + 1, 1)
                do_chunk(c0, 0)

                @pl.when(c0 + 1 < n_chunks)
                def _():
                    @pl.when(c0 + 2 < n_chunks)
                    def _():
                        start_dma(c0 + 2, 0)
                    do_chunk(c0 + 1, 1)
                return 0

            lax.fori_loop(0, (n_chunks + 1) // 2, two_chunks, 0)

            pltpu.sync_copy(
                p_acc.at[pl.ds(g * 128, 128)],
                p_hbm.at[pl.ds((out0 + g * 8) * 16, 128)])
            pltpu.sync_copy(
                z_acc.at[pl.ds(g * 128, 128)],
                z_hbm.at[pl.ds((out0 + g * 8) * 16, 128)])

        for g in range(n_groups):
            do_group(g)

    return sc_kernel(x, labels_pad, indexes)


def _tc_body(n_cols, n_blocks, blk_w, bt,
             xa_ref, xb_ref, lab_ref, y_ref, idx_ref,
             p_out, z_out, p_acc, z_acc):
    j = pl.program_id(0)

    @pl.when(j == 0)
    def _init():
        p_acc[...] = jnp.zeros_like(p_acc)
        z_acc[...] = jnp.zeros_like(z_acc)

    half = bt // 2
    col = j * blk_w + lax.broadcasted_iota(jnp.int32, (1, blk_w), 1)
    valid = col < n_cols
    lab = lab_ref[...]
    for x_ref, r0 in ((xa_ref, 0), (xb_ref, half)):
        rows = pl.ds(r0, half)
        e = jnp.exp(x_ref[...])
        e = jnp.where(valid & (col != idx_ref[rows, :]), e, 0.0)
        same = lab == y_ref[rows, :]
        z_acc[rows, :] += jnp.sum(e, axis=1, keepdims=True)
        p_acc[rows, :] += jnp.sum(jnp.where(same, e, 0.0), axis=1,
                                  keepdims=True)

    @pl.when(j == n_blocks - 1)
    def _emit():
        p_out[...] = p_acc[...]
        z_out[...] = z_acc[...]


def _tc_main(x_full, bt, labels2d, y2d, idx2d, blk_w=4096):
    n_cols = x_full.shape[1]
    half = bt // 2
    n_blocks = pl.cdiv(n_cols, blk_w)
    outp = jax.ShapeDtypeStruct((bt, 1), jnp.float32)
    body = functools.partial(_tc_body, n_cols, n_blocks, blk_w, bt)
    return pl.pallas_call(
        body,
        grid=(n_blocks,),
        in_specs=[
            pl.BlockSpec((half, blk_w), lambda j: (0, j)),
            pl.BlockSpec((half, blk_w), lambda j: (1, j)),
            pl.BlockSpec((1, blk_w), lambda j: (0, j)),
            pl.BlockSpec((bt, 1), lambda j: (0, 0)),
            pl.BlockSpec((bt, 1), lambda j: (0, 0)),
        ],
        out_specs=[
            pl.BlockSpec((bt, 1), lambda j: (0, 0)),
            pl.BlockSpec((bt, 1), lambda j: (0, 0)),
        ],
        out_shape=[outp, outp],
        scratch_shapes=[
            pltpu.VMEM((bt, 1), jnp.float32),
            pltpu.VMEM((bt, 1), jnp.float32),
        ],
        compiler_params=pltpu.CompilerParams(
            dimension_semantics=("arbitrary",),
        ),
    )(x_full, x_full, labels2d, y2d, idx2d)


def _tc_finalize(p_t, z_t, p16_s, z16_s, batch):
    out11 = jax.ShapeDtypeStruct((1, 1), jnp.float32)

    def stats(p, z):
        prob = p / z
        nzm = prob != 0.0
        logp = jnp.where(nzm, jnp.log(jnp.where(nzm, prob, 1.0)), 0.0)
        return jnp.sum(logp), jnp.min(p), jnp.sum(p)

    def body(pt_ref, zt_ref, ps_ref, zs_ref, loss_ref, min_ref, mean_ref):
        scale = 1.0 / math.exp(_MARGIN)
        pt = pt_ref[...] * scale
        zt = zt_ref[...] - pt_ref[...] + pt
        ps = jnp.sum(ps_ref[...], axis=1, keepdims=True) * scale
        zs = (jnp.sum(zs_ref[...], axis=1, keepdims=True)
              - jnp.sum(ps_ref[...], axis=1, keepdims=True) + ps)
        lt, mt, st = stats(pt, zt)
        ls, ms, ss = stats(ps, zs)
        loss_ref[...] = jnp.full((1, 1), -1.0 / batch) * (lt + ls)
        min_ref[...] = jnp.full((1, 1), 1.0) * jnp.minimum(mt, ms)
        mean_ref[...] = jnp.full((1, 1), 1.0 / batch) * (st + ss)

    return pl.pallas_call(
        body,
        out_shape=[out11, out11, out11],
    )(p_t, z_t, p16_s, z16_s)


def kernel(x, features, labels, indexes):
    del features
    batch, n_cols = x.shape
    s_rows = _SC_ROWS
    bt = batch - s_rows
    n_pad = _CW * ((n_cols + _CW - 1) // _CW)
    labels_pad = jnp.pad(labels, (0, n_pad - n_cols), constant_values=-1)
    y = jnp.take(labels, indexes)
    p_t, z_t = _tc_main(
        x, bt,
        labels.reshape(1, n_cols),
        y[:bt].reshape(bt, 1),
        indexes[:bt].reshape(bt, 1),
    )
    p16_s, z16_s = _sc_stream(x, labels_pad, indexes, bt, s_rows)
    loss, pmin, pmean = _tc_finalize(
        p_t, z_t,
        p16_s.reshape(s_rows, 16), z16_s.reshape(s_rows, 16),
        batch)
    return (loss[0, 0], pmin[0, 0], pmean[0, 0])
